# baseline (speedup 1.0000x reference)
import functools

import jax
import jax.numpy as jnp
from jax import lax
from jax.experimental import pallas as pl
from jax.experimental.pallas import tpu as pltpu
from jax.experimental.pallas import tpu_sc as plsc

N_NODES = 10000
N_EDGES = 320000
D = 128
NC_, NS = 2, 16
NW = NC_ * NS
EPW = N_EDGES // NW
CH = 40
NCHUNK = EPW // CH
NB = 8
LEAD = 4

assert EPW % CH == 0 and CH % 8 == 0 and LEAD <= NB - 2

_mesh = plsc.VectorSubcoreMesh(core_axis_name="c", subcore_axis_name="s")


def _one_gather(wx, wxhbm, idx_v, out, base, rbuf, gsem, hsem, wsem):
    def g(c, b):
        if b == NB - 1:
            pltpu.async_copy(
                wxhbm.at[idx_v.at[pl.ds(c * CH, CH)]], rbuf.at[b], hsem)
        else:
            pltpu.async_copy(
                wx.at[idx_v.at[pl.ds(c * CH, CH)]], rbuf.at[b], gsem)

    def gwait(b):
        if b == NB - 1:
            pltpu.make_async_copy(
                wxhbm.at[idx_v.at[pl.ds(0, CH)]], rbuf.at[b], hsem).wait()
        else:
            pltpu.make_async_copy(
                wx.at[idx_v.at[pl.ds(0, CH)]], rbuf.at[b], gsem).wait()

    def w(c, b):
        pltpu.async_copy(rbuf.at[b], out.at[pl.ds(base + c * CH, CH)], wsem)

    def wwait():
        pltpu.make_async_copy(
            rbuf.at[0], out.at[pl.ds(base, CH)], wsem).wait()

    def step(c, b):
        gwait(b)
        w(c, b)
        if c + LEAD < NCHUNK:
            if c + LEAD - NB >= 0:
                wwait()
            g(c + LEAD, (b + LEAD) % NB)

    def step_dyn(c, b):
        gwait(b)
        w(c, b)
        wwait()
        g(c + LEAD, (b + LEAD) % NB)

    for c in range(LEAD):
        g(c, c % NB)

    hi = NB + (NCHUNK - LEAD - NB) // NB * NB
    for c in range(NB):
        step(c, c % NB)

    def outer(i, carry):
        c0 = NB + i * NB
        for b in range(NB):
            step_dyn(c0 + b, b)
        return carry

    lax.fori_loop(0, (hi - NB) // NB, outer, 0)

    for c in range(hi, NCHUNK):
        step(c, c % NB)
    for _ in range(NB):
        wwait()


@functools.partial(
    pl.kernel,
    mesh=_mesh,
    out_type=(jax.ShapeDtypeStruct((N_EDGES, D), jnp.float32),
              jax.ShapeDtypeStruct((N_EDGES, D), jnp.float32)),
    scratch_types=[
        pltpu.VMEM((EPW,), jnp.int32),
        pltpu.VMEM((NB, CH, D), jnp.float32),
        pltpu.VMEM_SHARED((N_NODES, D), jnp.float32),
        pltpu.SemaphoreType.DMA,
        pltpu.SemaphoreType.DMA,
        pltpu.SemaphoreType.DMA,
    ],
)
def _gather_both(wx, eidx, out_i, out_j, idx_v, rbuf, shared,
                 gsem, hsem, wsem):
    wid = lax.axis_index("s") * NC_ + lax.axis_index("c")
    base = wid * EPW
    sid = lax.axis_index("s")
    rows = (N_NODES // NS) // 8 * 8
    off = sid * rows
    pltpu.sync_copy(wx.at[pl.ds(off, rows)], shared.at[pl.ds(off, rows)])

    @pl.when(sid == NS - 1)
    def _():
        rem_off = NS * rows
        pltpu.sync_copy(wx.at[pl.ds(rem_off, N_NODES - rem_off)],
                        shared.at[pl.ds(rem_off, N_NODES - rem_off)])

    pltpu.sync_copy(eidx.at[pl.ds(N_EDGES + base, EPW)], idx_v)
    plsc.subcore_barrier()
    _one_gather(shared, wx, idx_v, out_i, base, rbuf, gsem, hsem, wsem)
    pltpu.sync_copy(eidx.at[pl.ds(base, EPW)], idx_v)
    _one_gather(shared, wx, idx_v, out_j, base, rbuf, gsem, hsem, wsem)


def kernel(Wx, edge_index):
    eidx = edge_index.astype(jnp.int32).reshape(-1)
    return _gather_both(Wx, eidx)


# speedup vs baseline: 1.1130x; 1.1130x over previous
import functools

import jax
import jax.numpy as jnp
from jax import lax
from jax.experimental import pallas as pl
from jax.experimental.pallas import tpu as pltpu
from jax.experimental.pallas import tpu_sc as plsc

N_NODES = 10000
N_EDGES = 320000
D = 128
NC_, NS = 2, 16
NW = NC_ * NS
EPW = N_EDGES // NW
CH = 40
NCH = EPW // CH
NT = 2 * NCH
NB = 6
LEAD = 4

assert EPW % CH == 0 and CH % 8 == 0 and LEAD <= NB - 2

_mesh = plsc.VectorSubcoreMesh(core_axis_name="c", subcore_axis_name="s")


@functools.partial(
    pl.kernel,
    mesh=_mesh,
    out_type=(jax.ShapeDtypeStruct((N_EDGES, D), jnp.float32),
              jax.ShapeDtypeStruct((N_EDGES, D), jnp.float32)),
    scratch_types=[
        pltpu.VMEM((EPW,), jnp.int32),
        pltpu.VMEM((EPW,), jnp.int32),
        pltpu.VMEM((NB, CH, D), jnp.float32),
        pltpu.VMEM_SHARED((N_NODES, D), jnp.float32),
        pltpu.SemaphoreType.DMA,
        pltpu.SemaphoreType.DMA,
    ],
)
def _gather_both(wx, eidx, out_i, out_j, idx_i, idx_j, rbuf, shared,
                 gsem, wsem):
    wid = lax.axis_index("s") * NC_ + lax.axis_index("c")
    base = wid * EPW
    sid = lax.axis_index("s")
    rows = (N_NODES // NS) // 8 * 8
    off = sid * rows
    pltpu.sync_copy(wx.at[pl.ds(off, rows)], shared.at[pl.ds(off, rows)])

    @pl.when(sid == NS - 1)
    def _():
        rem_off = NS * rows
        pltpu.sync_copy(wx.at[pl.ds(rem_off, N_NODES - rem_off)],
                        shared.at[pl.ds(rem_off, N_NODES - rem_off)])

    pltpu.sync_copy(eidx.at[pl.ds(N_EDGES + base, EPW)], idx_i)
    pltpu.sync_copy(eidx.at[pl.ds(base, EPW)], idx_j)
    plsc.subcore_barrier()

    def g(t, b, idx, tb):
        pltpu.async_copy(shared.at[idx.at[pl.ds((t - tb) * CH, CH)]],
                         rbuf.at[b], gsem)

    def gwait(b):
        pltpu.make_async_copy(
            shared.at[idx_i.at[pl.ds(0, CH)]], rbuf.at[b], gsem).wait()

    def w(t, b, out, tb):
        pltpu.async_copy(rbuf.at[b],
                         out.at[pl.ds(base + (t - tb) * CH, CH)], wsem)

    def wwait():
        pltpu.make_async_copy(
            rbuf.at[0], out_i.at[pl.ds(base, CH)], wsem).wait()

    def py_g(t, b):
        if t < NCH:
            g(t, b, idx_i, 0)
        else:
            g(t, b, idx_j, NCH)

    def py_w(t, b):
        if t < NCH:
            w(t, b, out_i, 0)
        else:
            w(t, b, out_j, NCH)

    def py_step(t, b):
        gwait(b)
        py_w(t, b)
        if t + LEAD < NT:
            if t + LEAD - NB >= 0:
                wwait()
            py_g(t + LEAD, (b + LEAD) % NB)

    def steady(lo, n_blocks, out, idx, tb):
        def body(i, carry):
            t0 = lo + i * NB
            for k in range(NB):
                t = t0 + k
                gwait(k)
                w(t, k, out, tb)
                wwait()
                g(t + LEAD, (k + LEAD) % NB, idx, tb)
            return carry
        lax.fori_loop(0, n_blocks, body, 0)

    for t in range(LEAD):
        py_g(t, t % NB)

    HEAD = 2 * NB
    B1 = NCH - LEAD - (NCH - LEAD - HEAD) % NB
    B2 = ((NCH + LEAD + NB - 1) // NB) * NB
    B3 = B2 + (NT - LEAD - B2) // NB * NB
    assert HEAD % NB == 0 and B1 % NB == 0 and B2 % NB == 0

    for t in range(HEAD):
        py_step(t, t % NB)
    steady(HEAD, (B1 - HEAD) // NB, out_i, idx_i, 0)
    for t in range(B1, B2):
        py_step(t, t % NB)
    steady(B2, (B3 - B2) // NB, out_j, idx_j, NCH)
    for t in range(B3, NT):
        py_step(t, t % NB)
    for _ in range(NB):
        wwait()


def kernel(Wx, edge_index):
    eidx = edge_index.astype(jnp.int32).reshape(-1)
    return _gather_both(Wx, eidx)
